# Initial kernel scaffold; baseline (speedup 1.0000x reference)
import functools

import jax
import jax.numpy as jnp
from jax import lax
from jax.experimental import pallas as pl
from jax.experimental.pallas import tpu as pltpu
from jax.experimental.pallas import tpu_sc as plsc

N_NODES = 10000
N_EDGES = 320000
D_IN = 128
D_OUT = 128

NC = 2
NS = 16
NW = NC * NS

CHUNK = 80
EDGES_PER_TILE = N_EDGES // NW
NCHUNK = EDGES_PER_TILE // CHUNK
ROWS_PER_TILE = N_NODES // NS
CW = 16

_mesh = plsc.VectorSubcoreMesh(core_axis_name="c", subcore_axis_name="s")


@functools.partial(
    pl.kernel,
    mesh=_mesh,
    out_type=[
        jax.ShapeDtypeStruct((NC, N_NODES, D_IN), jnp.float32),
        jax.ShapeDtypeStruct((NC, N_NODES, CW), jnp.float32),
    ],
    scratch_types=[
        pltpu.VMEM((NCHUNK, CHUNK), jnp.int32),
        pltpu.VMEM((NCHUNK, CHUNK), jnp.int32),
        pltpu.VMEM((CHUNK, D_IN), jnp.float32),
        pltpu.VMEM((CHUNK, CW), jnp.float32),
        pltpu.VMEM_SHARED((N_NODES, D_IN), jnp.float32),
        pltpu.VMEM_SHARED((N_NODES, CW), jnp.float32),
        pltpu.SemaphoreType.DMA,
    ],
)
def _sc_aggregate(h_hbm, src_hbm, dst_hbm, zsum_hbm, zcnt_hbm, ones_hbm,
                  osum_hbm, ocnt_hbm,
                  src_v, dst_v, msg_v, ones_v, acc_s, acc_c, sem):
    cid = lax.axis_index("c")
    sid = lax.axis_index("s")
    wid = cid * NS + sid

    row0 = sid * ROWS_PER_TILE
    pltpu.sync_copy(zsum_hbm.at[pl.ds(row0, ROWS_PER_TILE)],
                    acc_s.at[pl.ds(row0, ROWS_PER_TILE)])
    pltpu.sync_copy(zcnt_hbm.at[pl.ds(row0, ROWS_PER_TILE)],
                    acc_c.at[pl.ds(row0, ROWS_PER_TILE)])
    pltpu.sync_copy(src_hbm.at[wid], src_v)
    pltpu.sync_copy(dst_hbm.at[wid], dst_v)
    pltpu.sync_copy(ones_hbm, ones_v)
    plsc.subcore_barrier()

    def body(j, carry):
        pltpu.async_copy(h_hbm.at[src_v.at[j]], msg_v, sem).wait()
        pltpu.sync_copy(msg_v, acc_s.at[dst_v.at[j]], add=True)
        pltpu.sync_copy(ones_v, acc_c.at[dst_v.at[j]], add=True)
        return carry

    lax.fori_loop(0, NCHUNK, body, 0)

    plsc.subcore_barrier()
    pltpu.sync_copy(acc_s.at[pl.ds(row0, ROWS_PER_TILE)],
                    osum_hbm.at[cid, pl.ds(row0, ROWS_PER_TILE)])
    pltpu.sync_copy(acc_c.at[pl.ds(row0, ROWS_PER_TILE)],
                    ocnt_hbm.at[cid, pl.ds(row0, ROWS_PER_TILE)])


def _tc_apply(h_ref, s_ref, c_ref, wt_ref, b_ref, o_ref):
    h = h_ref[...]
    s = s_ref[0] + s_ref[1]
    cnt = c_ref[0, :, 0:1] + c_ref[1, :, 0:1]
    cmean = s / jnp.maximum(cnt, 1.0)
    wt = wt_ref[...]
    y = (jnp.dot(h, wt[:D_IN], preferred_element_type=jnp.float32)
         + jnp.dot(cmean, wt[D_IN:], preferred_element_type=jnp.float32)
         + b_ref[...])
    n2 = jnp.sum(y * y, axis=1, keepdims=True)
    y = y / jnp.maximum(jnp.sqrt(n2), 1e-12)
    o_ref[...] = h + jnp.maximum(y, 0.0)


_BLK = 1000


def kernel(h, edge_index, W, b):
    src = edge_index[0].astype(jnp.int32).reshape(NW, NCHUNK, CHUNK)
    dst = edge_index[1].astype(jnp.int32).reshape(NW, NCHUNK, CHUNK)
    zsum = jnp.zeros((N_NODES, D_IN), jnp.float32)
    zcnt = jnp.zeros((N_NODES, CW), jnp.float32)
    ones = jnp.ones((CHUNK, CW), jnp.float32)
    psum, pcnt = _sc_aggregate(h, src, dst, zsum, zcnt, ones)

    grid = (N_NODES // _BLK,)
    out = pl.pallas_call(
        _tc_apply,
        grid=grid,
        in_specs=[
            pl.BlockSpec((_BLK, D_IN), lambda i: (i, 0)),
            pl.BlockSpec((NC, _BLK, D_IN), lambda i: (0, i, 0)),
            pl.BlockSpec((NC, _BLK, CW), lambda i: (0, i, 0)),
            pl.BlockSpec((2 * D_IN, D_OUT), lambda i: (0, 0)),
            pl.BlockSpec((1, D_OUT), lambda i: (0, 0)),
        ],
        out_specs=pl.BlockSpec((_BLK, D_OUT), lambda i: (i, 0)),
        out_shape=jax.ShapeDtypeStruct((N_NODES, D_OUT), jnp.float32),
    )(h, psum, pcnt, W.T, b.reshape(1, D_OUT))
    return out


# speedup vs baseline: 8.1154x; 8.1154x over previous
import functools

import jax
import jax.numpy as jnp
from jax import lax
from jax.experimental import pallas as pl
from jax.experimental.pallas import tpu as pltpu
from jax.experimental.pallas import tpu_sc as plsc

N_NODES = 10000
N_EDGES = 320000
D_IN = 128
D_OUT = 128

NC = 2
NS = 16
NW = NC * NS

CHUNK = 80
EDGES_PER_TILE = N_EDGES // NW
NCHUNK = EDGES_PER_TILE // CHUNK
NGRP = 5
GS = NCHUNK // NGRP
NROW = 10240
ROWS_PER_TILE = NROW // NS
CW = 16

_mesh = plsc.VectorSubcoreMesh(core_axis_name="c", subcore_axis_name="s")


@functools.partial(
    pl.kernel,
    mesh=_mesh,
    compiler_params=pltpu.CompilerParams(use_tc_tiling_on_sc=False),
    out_type=[
        jax.ShapeDtypeStruct((NC, NROW, D_IN), jnp.float32),
        jax.ShapeDtypeStruct((NC, NROW, CW), jnp.float32),
    ],
    scratch_types=[
        pltpu.VMEM((GS, CHUNK), jnp.int32),
        pltpu.VMEM((GS, CHUNK), jnp.int32),
        pltpu.VMEM((CHUNK, D_IN), jnp.float32),
        pltpu.VMEM((CHUNK, CW), jnp.float32),
        pltpu.VMEM_SHARED((NROW, D_IN), jnp.float32),
        pltpu.VMEM_SHARED((NROW, CW), jnp.float32),
        pltpu.SemaphoreType.DMA,
    ],
)
def _sc_aggregate(h_hbm, src_hbm, dst_hbm, zsum_hbm, zcnt_hbm, ones_hbm,
                  osum_hbm, ocnt_hbm,
                  src_v, dst_v, msg_v, ones_v, acc_s, acc_c, sem):
    cid = lax.axis_index("c")
    sid = lax.axis_index("s")
    wid = cid * NS + sid

    row0 = sid * ROWS_PER_TILE
    pltpu.sync_copy(zsum_hbm, msg_v)
    pltpu.sync_copy(zcnt_hbm, ones_v)
    for k in range(ROWS_PER_TILE // CHUNK):
        r = row0 + k * CHUNK
        pltpu.sync_copy(msg_v, acc_s.at[pl.ds(r, CHUNK)])
        pltpu.sync_copy(ones_v, acc_c.at[pl.ds(r, CHUNK)])
    pltpu.sync_copy(ones_hbm, ones_v)
    plsc.subcore_barrier()

    def group(g, carry):
        pltpu.sync_copy(src_hbm.at[wid, g], src_v)
        pltpu.sync_copy(dst_hbm.at[wid, g], dst_v)

        def body(j, carry2):
            pltpu.async_copy(h_hbm.at[src_v.at[j]], msg_v, sem).wait()
            pltpu.sync_copy(msg_v, acc_s.at[dst_v.at[j]], add=True)
            pltpu.sync_copy(ones_v, acc_c.at[dst_v.at[j]], add=True)
            return carry2

        return lax.fori_loop(0, GS, body, carry)

    lax.fori_loop(0, NGRP, group, 0)

    plsc.subcore_barrier()
    for k in range(ROWS_PER_TILE // CHUNK):
        r = row0 + k * CHUNK
        pltpu.sync_copy(acc_s.at[pl.ds(r, CHUNK)], msg_v)
        pltpu.sync_copy(msg_v, osum_hbm.at[cid, pl.ds(r, CHUNK)])
        pltpu.sync_copy(acc_c.at[pl.ds(r, CHUNK)], ones_v)
        pltpu.sync_copy(ones_v, ocnt_hbm.at[cid, pl.ds(r, CHUNK)])


def _tc_apply(h_ref, s_ref, c_ref, wt_ref, b_ref, o_ref):
    h = h_ref[...]
    s = s_ref[0] + s_ref[1]
    cnt = c_ref[0, :, 0:1] + c_ref[1, :, 0:1]
    cmean = s / jnp.maximum(cnt, 1.0)
    wt = wt_ref[...]
    y = (jnp.dot(h, wt[:D_IN], preferred_element_type=jnp.float32)
         + jnp.dot(cmean, wt[D_IN:], preferred_element_type=jnp.float32)
         + b_ref[...])
    n2 = jnp.sum(y * y, axis=1, keepdims=True)
    y = y / jnp.maximum(jnp.sqrt(n2), 1e-12)
    o_ref[...] = h + jnp.maximum(y, 0.0)


_BLK = 1000


def kernel(h, edge_index, W, b):
    src = edge_index[0].astype(jnp.int32).reshape(NW, NGRP, GS, CHUNK)
    dst = edge_index[1].astype(jnp.int32).reshape(NW, NGRP, GS, CHUNK)
    zsum = jnp.zeros((CHUNK, D_IN), jnp.float32)
    zcnt = jnp.zeros((CHUNK, CW), jnp.float32)
    ones = jnp.ones((CHUNK, CW), jnp.float32)
    psum, pcnt = _sc_aggregate(h, src, dst, zsum, zcnt, ones)

    grid = (N_NODES // _BLK,)
    out = pl.pallas_call(
        _tc_apply,
        grid=grid,
        in_specs=[
            pl.BlockSpec((_BLK, D_IN), lambda i: (i, 0)),
            pl.BlockSpec((NC, _BLK, D_IN), lambda i: (0, i, 0)),
            pl.BlockSpec((NC, _BLK, CW), lambda i: (0, i, 0)),
            pl.BlockSpec((2 * D_IN, D_OUT), lambda i: (0, 0)),
            pl.BlockSpec((1, D_OUT), lambda i: (0, 0)),
        ],
        out_specs=pl.BlockSpec((_BLK, D_OUT), lambda i: (i, 0)),
        out_shape=jax.ShapeDtypeStruct((N_NODES, D_OUT), jnp.float32),
    )(h, psum, pcnt, W.T, b.reshape(1, D_OUT))
    return out


# speedup vs baseline: 8.9117x; 1.0981x over previous
import functools

import jax
import jax.numpy as jnp
from jax import lax
from jax.experimental import pallas as pl
from jax.experimental.pallas import tpu as pltpu
from jax.experimental.pallas import tpu_sc as plsc

N_NODES = 10000
N_EDGES = 320000
D_IN = 128
D_OUT = 128

NC = 2
NS = 16
NW = NC * NS

CHUNK = 80
EDGES_PER_TILE = N_EDGES // NW
NCHUNK = EDGES_PER_TILE // CHUNK
NGRP = 5
GS = NCHUNK // NGRP
NROW = 10240
ROWS_PER_TILE = NROW // NS
CW = 16
DA = D_IN + CW

_mesh = plsc.VectorSubcoreMesh(core_axis_name="c", subcore_axis_name="s")


@functools.partial(
    pl.kernel,
    mesh=_mesh,
    compiler_params=pltpu.CompilerParams(use_tc_tiling_on_sc=False),
    out_type=jax.ShapeDtypeStruct((NC, NROW, DA), jnp.float32),
    scratch_types=[
        pltpu.VMEM((GS, CHUNK), jnp.int32),
        pltpu.VMEM((GS, CHUNK), jnp.int32),
        pltpu.VMEM((CHUNK, DA), jnp.float32),
        pltpu.VMEM((CHUNK, DA), jnp.float32),
        pltpu.VMEM_SHARED((NROW, DA), jnp.float32),
        pltpu.SemaphoreType.DMA,
        pltpu.SemaphoreType.DMA,
        pltpu.SemaphoreType.DMA,
        pltpu.SemaphoreType.DMA,
    ],
)
def _sc_aggregate(h_hbm, src_hbm, dst_hbm, zrow_hbm, o_hbm,
                  src_v, dst_v, msg0, msg1, acc,
                  g_sem0, g_sem1, s_sem0, s_sem1):
    cid = lax.axis_index("c")
    sid = lax.axis_index("s")
    wid = cid * NS + sid
    msg = (msg0, msg1)
    g_sem = (g_sem0, g_sem1)
    s_sem = (s_sem0, s_sem1)

    def fire_gather(b, j):
        pltpu.async_copy(h_hbm.at[src_v.at[j]], msg[b], g_sem[b])

    def wait_gather(b):
        pltpu.make_async_copy(h_hbm.at[src_v.at[0]], msg[b], g_sem[b]).wait()

    def fire_scatter(b, j):
        pltpu.async_copy(msg[b], acc.at[dst_v.at[j]], s_sem[b], add=True)

    def wait_scatter(b):
        pltpu.make_async_copy(msg[b], acc.at[dst_v.at[0]], s_sem[b]).wait()

    row0 = sid * ROWS_PER_TILE
    pltpu.sync_copy(zrow_hbm, msg0)
    for k in range(ROWS_PER_TILE // CHUNK):
        pltpu.sync_copy(msg0, acc.at[pl.ds(row0 + k * CHUNK, CHUNK)])
    plsc.subcore_barrier()

    for g in range(NGRP):
        pltpu.sync_copy(src_hbm.at[wid, g], src_v)
        pltpu.sync_copy(dst_hbm.at[wid, g], dst_v)
        fire_gather(0, 0)
        fire_gather(1, 1)

        @pl.loop(0, GS - 4, step=2)
        def _(j):
            wait_gather(0)
            fire_scatter(0, j)
            wait_gather(1)
            fire_scatter(1, j + 1)
            wait_scatter(0)
            fire_gather(0, j + 2)
            wait_scatter(1)
            fire_gather(1, j + 3)

        j = GS - 3
        wait_gather(0)
        fire_scatter(0, j)
        wait_gather(1)
        fire_scatter(1, j + 1)
        wait_scatter(0)
        fire_gather(0, j + 2)
        wait_gather(0)
        fire_scatter(0, j + 2)
        wait_scatter(1)
        wait_scatter(0)

    plsc.subcore_barrier()
    for k in range(ROWS_PER_TILE // CHUNK):
        r = row0 + k * CHUNK
        pltpu.sync_copy(acc.at[pl.ds(r, CHUNK)], msg0)
        pltpu.sync_copy(msg0, o_hbm.at[cid, pl.ds(r, CHUNK)])


def _tc_apply(h_ref, p_ref, wt_ref, b_ref, o_ref):
    h = h_ref[...]
    p = p_ref[0] + p_ref[1]
    s = p[:, :D_IN]
    cnt = p[:, D_IN:D_IN + 1]
    cmean = s / jnp.maximum(cnt, 1.0)
    wt = wt_ref[...]
    y = (jnp.dot(h, wt[:D_IN], preferred_element_type=jnp.float32)
         + jnp.dot(cmean, wt[D_IN:], preferred_element_type=jnp.float32)
         + b_ref[...])
    n2 = jnp.sum(y * y, axis=1, keepdims=True)
    y = y / jnp.maximum(jnp.sqrt(n2), 1e-12)
    o_ref[...] = h + jnp.maximum(y, 0.0)


_BLK = 1000


def kernel(h, edge_index, W, b):
    src = edge_index[0].astype(jnp.int32).reshape(NW, NGRP, GS, CHUNK)
    dst = edge_index[1].astype(jnp.int32).reshape(NW, NGRP, GS, CHUNK)
    h_aug = jnp.concatenate([h, jnp.ones((N_NODES, CW), jnp.float32)], axis=1)
    zrow = jnp.zeros((CHUNK, DA), jnp.float32)
    part = _sc_aggregate(h_aug, src, dst, zrow)

    grid = (N_NODES // _BLK,)
    out = pl.pallas_call(
        _tc_apply,
        grid=grid,
        in_specs=[
            pl.BlockSpec((_BLK, D_IN), lambda i: (i, 0)),
            pl.BlockSpec((NC, _BLK, DA), lambda i: (0, i, 0)),
            pl.BlockSpec((2 * D_IN, D_OUT), lambda i: (0, 0)),
            pl.BlockSpec((1, D_OUT), lambda i: (0, 0)),
        ],
        out_specs=pl.BlockSpec((_BLK, D_OUT), lambda i: (i, 0)),
        out_shape=jax.ShapeDtypeStruct((N_NODES, D_OUT), jnp.float32),
    )(h, part, W.T, b.reshape(1, D_OUT))
    return out
